# baseline (speedup 1.0000x reference)
import functools

import jax
import jax.numpy as jnp
from jax import lax
from jax.experimental import pallas as pl
from jax.experimental.pallas import tpu as pltpu
from jax.experimental.pallas import tpu_sc as plsc

NTOK = 4096
S = 50
D = 128
NSPLIT = 4
QTOK = NTOK // NSPLIT
NC, NS = 2, 16
NW = NC * NS
TOK_PER_W = QTOK // NW
ROWS_PER_W = TOK_PER_W * S
TCHUNK = 8
CR = TCHUNK * S
G = 80
NG = CR // G
NCHUNK = TOK_PER_W // TCHUNK
NBUF = 2
NGROUP = NCHUNK // NBUF

_mesh = plsc.VectorSubcoreMesh(core_axis_name="c", subcore_axis_name="s")


@functools.partial(
    pl.kernel,
    mesh=_mesh,
    out_type=jax.ShapeDtypeStruct((QTOK, S, D), jnp.float32),
    scratch_types=[
        pltpu.VMEM((ROWS_PER_W,), jnp.int32),
        pltpu.VMEM((NBUF * CR, D), jnp.float32),
        pltpu.SemaphoreType.DMA,
        pltpu.SemaphoreType.DMA,
    ],
)
def _emb_lookup(idx_hbm, table_hbm, out_hbm, idx_v, rows_v, sem0, sem1):
    sems = [sem0, sem1]
    wid = lax.axis_index("s") * NC + lax.axis_index("c")
    row_base = wid * ROWS_PER_W
    tok_base = wid * TOK_PER_W

    pltpu.sync_copy(idx_hbm.at[pl.ds(row_base, ROWS_PER_W)], idx_v)

    def start_chunk(c, b):
        for j in range(NG):
            off = j * G
            pltpu.async_copy(
                table_hbm.at[idx_v.at[pl.ds(c * CR + off, G)]],
                rows_v.at[pl.ds(b * CR + off, G)],
                sems[b],
            )

    for b in range(NBUF):
        start_chunk(b, b)

    def group(g, carry):
        for b in range(NBUF):
            c = g * NBUF + b
            pltpu.make_async_copy(
                table_hbm.at[pl.ds(0, CR)],
                rows_v.at[pl.ds(b * CR, CR)],
                sems[b],
            ).wait()
            for t in range(TCHUNK):
                pltpu.sync_copy(
                    rows_v.at[pl.ds(b * CR + t * S, S)],
                    out_hbm.at[tok_base + c * TCHUNK + t],
                )
            nxt = c + NBUF

            @pl.when(nxt < NCHUNK)
            def _():
                start_chunk(nxt, b)

        return carry

    lax.fori_loop(0, NGROUP, group, 0)


def kernel(token_ids, embedding):
    flat = token_ids.reshape(-1).astype(jnp.int32)
    parts = [
        _emb_lookup(lax.slice(flat, (q * QTOK * S,), ((q + 1) * QTOK * S,)), embedding)
        for q in range(NSPLIT)
    ]
    return jnp.concatenate(parts, axis=0)


# speedup vs baseline: 1.0386x; 1.0386x over previous
import functools

import jax
import jax.numpy as jnp
from jax import lax
from jax.experimental import pallas as pl
from jax.experimental.pallas import tpu as pltpu
from jax.experimental.pallas import tpu_sc as plsc

NTOK = 4096
S = 50
D = 128
NSPLIT = 4
QTOK = NTOK // NSPLIT
NC, NS = 2, 16
NW = NC * NS
TOK_PER_W = QTOK // NW
ROWS_PER_W = TOK_PER_W * S
TCHUNK = 8
CR = TCHUNK * S
G = 80
NG = CR // G
NCHUNK = TOK_PER_W // TCHUNK
NBUF = 2
NGROUP = NCHUNK // NBUF

_mesh = plsc.VectorSubcoreMesh(core_axis_name="c", subcore_axis_name="s")


@functools.partial(
    pl.kernel,
    mesh=_mesh,
    out_type=jax.ShapeDtypeStruct((QTOK, S, D), jnp.float32),
    scratch_types=[
        pltpu.VMEM((ROWS_PER_W,), jnp.int32),
        pltpu.VMEM((NBUF * CR, D), jnp.float32),
        pltpu.SemaphoreType.DMA,
        pltpu.SemaphoreType.DMA,
    ],
)
def _emb_lookup(idx_hbm, table_hbm, out_hbm, idx_v, rows_v, sem0, sem1):
    sems = [sem0, sem1]
    wid = lax.axis_index("s") * NC + lax.axis_index("c")
    row_base = wid * ROWS_PER_W
    tok_base = wid * TOK_PER_W

    pltpu.sync_copy(idx_hbm.at[pl.ds(row_base, ROWS_PER_W)], idx_v)

    def start_chunk(c, b):
        for j in range(NG):
            off = j * G
            pltpu.async_copy(
                table_hbm.at[idx_v.at[pl.ds(c * CR + off, G)]],
                rows_v.at[pl.ds(b * CR + off, G)],
                sems[b],
            )

    for b in range(NBUF):
        start_chunk(b, b)

    def group(g, carry):
        for b in range(NBUF):
            c = g * NBUF + b
            pltpu.make_async_copy(
                table_hbm.at[pl.ds(0, CR)],
                rows_v.at[pl.ds(b * CR, CR)],
                sems[b],
            ).wait()
            for t in range(TCHUNK):
                pltpu.sync_copy(
                    rows_v.at[pl.ds(b * CR + t * S, S)],
                    out_hbm.at[tok_base + c * TCHUNK + t],
                )
            nxt = c + NBUF

            @pl.when(nxt < NCHUNK)
            def _():
                start_chunk(nxt, b)

        return carry

    lax.fori_loop(0, NGROUP, group, 0)


def kernel(token_ids, embedding):
    flat = token_ids.reshape(-1).astype(jnp.int32)
    out = jnp.zeros((NTOK, S, D), jnp.float32)
    for q in range(NSPLIT):
        part = _emb_lookup(
            lax.slice(flat, (q * QTOK * S,), ((q + 1) * QTOK * S,)), embedding
        )
        out = lax.dynamic_update_slice(out, part, (q * QTOK, 0, 0))
    return out


# speedup vs baseline: 1.8027x; 1.7356x over previous
import functools

import jax
import jax.numpy as jnp
from jax import lax
from jax.experimental import pallas as pl
from jax.experimental.pallas import tpu as pltpu
from jax.experimental.pallas import tpu_sc as plsc
from jax.experimental.layout import Format, Layout
from jax.sharding import SingleDeviceSharding

NTOK = 4096
S = 50
D = 128
NC, NS = 2, 16
NW = NC * NS
TOK_PER_W = NTOK // NW
ROWS_PER_W = TOK_PER_W * S
TCHUNK = 8
CR = TCHUNK * S
G = 80
NG = CR // G
NCHUNK = TOK_PER_W // TCHUNK
NBUF = 2
NGROUP = NCHUNK // NBUF

_mesh = plsc.VectorSubcoreMesh(core_axis_name="c", subcore_axis_name="s")


def _emb_lookup_body(idx_hbm, table_hbm, out_hbm, idx_v, rows_v, sem0, sem1):
    sems = [sem0, sem1]
    wid = lax.axis_index("s") * NC + lax.axis_index("c")
    row_base = wid * ROWS_PER_W
    tok_base = wid * TOK_PER_W

    pltpu.sync_copy(idx_hbm.at[pl.ds(row_base, ROWS_PER_W)], idx_v)

    def start_chunk(c, b):
        for j in range(NG):
            off = j * G
            pltpu.async_copy(
                table_hbm.at[idx_v.at[pl.ds(c * CR + off, G)]],
                rows_v.at[pl.ds(b * CR + off, G)],
                sems[b],
            )

    for b in range(NBUF):
        start_chunk(b, b)

    def group(g, carry):
        for b in range(NBUF):
            c = g * NBUF + b
            pltpu.make_async_copy(
                table_hbm.at[pl.ds(0, CR)],
                rows_v.at[pl.ds(b * CR, CR)],
                sems[b],
            ).wait()
            for t in range(TCHUNK):
                pltpu.sync_copy(
                    rows_v.at[pl.ds(b * CR + t * S, S)],
                    out_hbm.at[tok_base + c * TCHUNK + t],
                )
            nxt = c + NBUF

            @pl.when(nxt < NCHUNK)
            def _():
                start_chunk(nxt, b)

        return carry

    lax.fori_loop(0, NGROUP, group, 0)


def _make_lookup(out_type):
    return pl.kernel(
        _emb_lookup_body,
        mesh=_mesh,
        out_type=out_type,
        scratch_types=[
            pltpu.VMEM((ROWS_PER_W,), jnp.int32),
            pltpu.VMEM((NBUF * CR, D), jnp.float32),
            pltpu.SemaphoreType.DMA,
            pltpu.SemaphoreType.DMA,
        ],
    )


def kernel(token_ids, embedding):
    flat = token_ids.reshape(-1).astype(jnp.int32)
    out_type = jax.ShapeDtypeStruct((NTOK, S, D), jnp.float32)
    try:
        fmt = Format(
            Layout((0, 1, 2), tiling=((8, 128),)),
            SingleDeviceSharding(jax.devices()[0]),
        )
        out_type = out_type.update(format=fmt)
    except Exception:
        pass
    return _make_lookup(out_type)(flat, embedding)


# speedup vs baseline: 1.8076x; 1.0027x over previous
import functools

import jax
import jax.numpy as jnp
from jax import lax
from jax.experimental import pallas as pl
from jax.experimental.pallas import tpu as pltpu
from jax.experimental.pallas import tpu_sc as plsc

NTOK = 4096
S = 50
D = 128
NC, NS = 2, 16
NW = NC * NS
TOK_PER_W = NTOK // NW
ROWS_PER_W = TOK_PER_W * S
TCHUNK = 4
CR = TCHUNK * S
G = 40
NG = CR // G
NCHUNK = TOK_PER_W // TCHUNK
NBUF = 4
NGROUP = NCHUNK // NBUF

_mesh = plsc.VectorSubcoreMesh(core_axis_name="c", subcore_axis_name="s")


@functools.partial(
    pl.kernel,
    mesh=_mesh,
    out_type=jax.ShapeDtypeStruct((NTOK, S, D), jnp.float32),
    scratch_types=[
        pltpu.VMEM((ROWS_PER_W,), jnp.int32),
        pltpu.VMEM((NBUF * CR, D), jnp.float32),
    ]
    + [pltpu.SemaphoreType.DMA] * (2 * NBUF),
)
def _emb_lookup(idx_hbm, table_hbm, out_hbm, idx_v, rows_v, *sems):
    gsems, wsems = sems[:NBUF], sems[NBUF:]
    wid = lax.axis_index("s") * NC + lax.axis_index("c")
    row_base = wid * ROWS_PER_W
    tok_base = wid * TOK_PER_W

    pltpu.sync_copy(idx_hbm.at[pl.ds(row_base, ROWS_PER_W)], idx_v)

    def start_chunk(c, b):
        for j in range(NG):
            off = j * G
            pltpu.async_copy(
                table_hbm.at[idx_v.at[pl.ds(c * CR + off, G)]],
                rows_v.at[pl.ds(b * CR + off, G)],
                gsems[b],
            )

    def chunk_wait(sem, b):
        pltpu.make_async_copy(
            table_hbm.at[pl.ds(0, CR)], rows_v.at[pl.ds(b * CR, CR)], sem
        ).wait()

    for b in range(NBUF):
        start_chunk(b, b)

    def group(g, carry):
        for b in range(NBUF):
            c = g * NBUF + b
            chunk_wait(gsems[b], b)
            for t in range(TCHUNK):
                pltpu.async_copy(
                    rows_v.at[pl.ds(b * CR + t * S, S)],
                    out_hbm.at[tok_base + c * TCHUNK + t],
                    wsems[b],
                )
            j = c + NBUF - 1
            b2 = (b + NBUF - 1) % NBUF

            @pl.when(jnp.logical_and(j >= NBUF, j < NCHUNK))
            def _():
                chunk_wait(wsems[b2], b2)
                start_chunk(j, b2)

        return carry

    lax.fori_loop(0, NGROUP, group, 0)

    for b in range(NBUF):
        chunk_wait(wsems[b], b)


def kernel(token_ids, embedding):
    flat = token_ids.reshape(-1).astype(jnp.int32)
    return _emb_lookup(flat, embedding)
